# baseline (speedup 1.0000x reference)
import jax
import jax.numpy as jnp
from jax.experimental import pallas as pl
from jax.experimental.pallas import tpu as pltpu

_BR = 512


def _boxcox_block(tab_ref, x_ref, g_ref, o_ref):
    xv = x_ref[...]
    g = g_ref[...]
    lx = jnp.log(xv)
    lam = jnp.full(xv.shape, tab_ref[0, 0], dtype=jnp.float32)
    inv = jnp.full(xv.shape, tab_ref[1, 0], dtype=jnp.float32)
    for k in range(1, 8):
        m = g == (k + 1)
        lam = jnp.where(m, tab_ref[0, k], lam)
        inv = jnp.where(m, tab_ref[1, k], inv)
    pow_branch = (jnp.exp(lam * lx) - 1.0) * inv
    o_ref[...] = jnp.where(jnp.abs(lam) < 0.01, lx, pow_branch)


def kernel(x, group, lmbdas):
    R, C = x.shape
    inv = jnp.where(jnp.abs(lmbdas) < 0.01, jnp.float32(1.0), 1.0 / lmbdas)
    tab = jnp.stack([lmbdas, inv])
    grid = (R // _BR,)
    return pl.pallas_call(
        _boxcox_block,
        grid=grid,
        in_specs=[
            pl.BlockSpec((2, 8), lambda i: (0, 0)),
            pl.BlockSpec((_BR, C), lambda i: (i, 0)),
            pl.BlockSpec((_BR, C), lambda i: (i, 0)),
        ],
        out_specs=pl.BlockSpec((_BR, C), lambda i: (i, 0)),
        out_shape=jax.ShapeDtypeStruct((R, C), jnp.float32),
        compiler_params=pltpu.CompilerParams(
            dimension_semantics=("parallel",),
        ),
    )(tab, x, group)


# speedup vs baseline: 1.1161x; 1.1161x over previous
import jax
import jax.numpy as jnp
from jax.experimental import pallas as pl
from jax.experimental.pallas import tpu as pltpu

_BR = 512


def _boxcox_block(tab_ref, x_ref, g_ref, o_ref):
    xv = x_ref[...]
    g = g_ref[...]
    lx = jnp.log(xv)
    lam = jnp.full(xv.shape, tab_ref[0], dtype=jnp.float32)
    inv = jnp.full(xv.shape, tab_ref[8], dtype=jnp.float32)
    for k in range(1, 8):
        m = g == (k + 1)
        lam = jnp.where(m, tab_ref[k], lam)
        inv = jnp.where(m, tab_ref[8 + k], inv)
    pow_branch = (jnp.exp(lam * lx) - 1.0) * inv
    o_ref[...] = jnp.where(jnp.abs(lam) < 0.01, lx, pow_branch)


def kernel(x, group, lmbdas):
    R, C = x.shape
    inv = jnp.where(jnp.abs(lmbdas) < 0.01, jnp.float32(1.0), 1.0 / lmbdas)
    tab = jnp.concatenate([lmbdas, inv])
    grid = (R // _BR,)
    return pl.pallas_call(
        _boxcox_block,
        grid=grid,
        in_specs=[
            pl.BlockSpec(memory_space=pltpu.SMEM),
            pl.BlockSpec((_BR, C), lambda i: (i, 0)),
            pl.BlockSpec((_BR, C), lambda i: (i, 0)),
        ],
        out_specs=pl.BlockSpec((_BR, C), lambda i: (i, 0)),
        out_shape=jax.ShapeDtypeStruct((R, C), jnp.float32),
        compiler_params=pltpu.CompilerParams(
            dimension_semantics=("parallel",),
        ),
    )(tab, x, group)


# speedup vs baseline: 1.3004x; 1.1651x over previous
import jax
import jax.numpy as jnp
from jax.experimental import pallas as pl
from jax.experimental.pallas import tpu as pltpu

_BR = 2048


def _boxcox_block(tab_ref, x_ref, g_ref, o_ref):
    xv = x_ref[...]
    g = g_ref[...]
    lx = jnp.log(xv)
    lam = jnp.full(xv.shape, tab_ref[0], dtype=jnp.float32)
    inv = jnp.full(xv.shape, tab_ref[8], dtype=jnp.float32)
    for k in range(1, 8):
        m = g == (k + 1)
        lam = jnp.where(m, tab_ref[k], lam)
        inv = jnp.where(m, tab_ref[8 + k], inv)
    pow_branch = (jnp.exp(lam * lx) - 1.0) * inv
    o_ref[...] = jnp.where(jnp.abs(lam) < 0.01, lx, pow_branch)


def kernel(x, group, lmbdas):
    R, C = x.shape
    inv = jnp.where(jnp.abs(lmbdas) < 0.01, jnp.float32(1.0), 1.0 / lmbdas)
    tab = jnp.concatenate([lmbdas, inv])
    grid = (R // _BR,)
    return pl.pallas_call(
        _boxcox_block,
        grid=grid,
        in_specs=[
            pl.BlockSpec(memory_space=pltpu.SMEM),
            pl.BlockSpec((_BR, C), lambda i: (i, 0)),
            pl.BlockSpec((_BR, C), lambda i: (i, 0)),
        ],
        out_specs=pl.BlockSpec((_BR, C), lambda i: (i, 0)),
        out_shape=jax.ShapeDtypeStruct((R, C), jnp.float32),
        compiler_params=pltpu.CompilerParams(
            dimension_semantics=("parallel",),
        ),
    )(tab, x, group)


# speedup vs baseline: 1.3444x; 1.0339x over previous
import jax
import jax.numpy as jnp
from jax.experimental import pallas as pl
from jax.experimental.pallas import tpu as pltpu

_CR = 512
_NBUF = 4


def _compute(tab_ref, xv, g):
    lx = jnp.log(xv)
    lam = jnp.full(xv.shape, tab_ref[0], dtype=jnp.float32)
    inv = jnp.full(xv.shape, tab_ref[8], dtype=jnp.float32)
    for k in range(1, 8):
        m = g == (k + 1)
        lam = jnp.where(m, tab_ref[k], lam)
        inv = jnp.where(m, tab_ref[8 + k], inv)
    pow_branch = (jnp.exp(lam * lx) - 1.0) * inv
    return jnp.where(jnp.abs(lam) < 0.01, lx, pow_branch)


def _make_body(R, C):
    nchunks = R // _CR
    ngroups = nchunks // _NBUF

    def body(tab_ref, x_hbm, g_hbm, o_hbm, xb, gb, ob, xs, gs, osem):
        def start_in(b, c):
            pltpu.make_async_copy(x_hbm.at[pl.ds(c * _CR, _CR)], xb.at[b], xs.at[b]).start()
            pltpu.make_async_copy(g_hbm.at[pl.ds(c * _CR, _CR)], gb.at[b], gs.at[b]).start()

        def wait_in(b, c):
            pltpu.make_async_copy(x_hbm.at[pl.ds(c * _CR, _CR)], xb.at[b], xs.at[b]).wait()
            pltpu.make_async_copy(g_hbm.at[pl.ds(c * _CR, _CR)], gb.at[b], gs.at[b]).wait()

        def start_out(b, c):
            pltpu.make_async_copy(ob.at[b], o_hbm.at[pl.ds(c * _CR, _CR)], osem.at[b]).start()

        def wait_out(b, c):
            pltpu.make_async_copy(ob.at[b], o_hbm.at[pl.ds(c * _CR, _CR)], osem.at[b]).wait()

        for b in range(_NBUF):
            start_in(b, b)

        def gbody(gi, carry):
            for b in range(_NBUF):
                c = gi * _NBUF + b
                wait_in(b, c)

                @pl.when(gi > 0)
                def _():
                    wait_out(b, c - _NBUF)

                ob[b] = _compute(tab_ref, xb[b], gb[b])
                start_out(b, c)

                @pl.when(gi + 1 < ngroups)
                def _():
                    start_in(b, c + _NBUF)

            return carry

        jax.lax.fori_loop(0, ngroups, gbody, 0)

        for b in range(_NBUF):
            wait_out(b, (ngroups - 1) * _NBUF + b)

    return body


def kernel(x, group, lmbdas):
    R, C = x.shape
    inv = jnp.where(jnp.abs(lmbdas) < 0.01, jnp.float32(1.0), 1.0 / lmbdas)
    tab = jnp.concatenate([lmbdas, inv])
    return pl.pallas_call(
        _make_body(R, C),
        in_specs=[
            pl.BlockSpec(memory_space=pltpu.SMEM),
            pl.BlockSpec(memory_space=pltpu.HBM),
            pl.BlockSpec(memory_space=pltpu.HBM),
        ],
        out_specs=pl.BlockSpec(memory_space=pltpu.HBM),
        out_shape=jax.ShapeDtypeStruct((R, C), jnp.float32),
        scratch_shapes=[
            pltpu.VMEM((_NBUF, _CR, C), jnp.float32),
            pltpu.VMEM((_NBUF, _CR, C), jnp.int32),
            pltpu.VMEM((_NBUF, _CR, C), jnp.float32),
            pltpu.SemaphoreType.DMA((_NBUF,)),
            pltpu.SemaphoreType.DMA((_NBUF,)),
            pltpu.SemaphoreType.DMA((_NBUF,)),
        ],
    )(tab, x, group)
